# Initial kernel scaffold; baseline (speedup 1.0000x reference)
import jax
import jax.numpy as jnp
from jax.experimental import pallas as pl


def kernel(x, edge_index, W1, b1, base_w, spline_w, W2, b2, fc_w, fc_b):
    raise NotImplementedError("write your pallas kernel here")


# speedup vs baseline: 13.2300x; 13.2300x over previous
import jax
import jax.numpy as jnp
from jax import lax
from jax.experimental import pallas as pl
from jax.experimental.pallas import tpu as pltpu
from jax.experimental.pallas import tpu_sc as plsc

N = 10000
E = 320000
D = 128
H = 128
C = 64

NC = 2
NS = 16
NW = NC * NS
HH = H // NC

CHUNK = 80
NCH_DEG = (E // NW) // CHUNK
NCH_CONV = (E // NS) // CHUNK

NP = 10240
RPS = NP // NS
ZROWS = 128

BM = 400


def _sc_mesh():
    return plsc.VectorSubcoreMesh(core_axis_name="c", subcore_axis_name="s")


def _sc_degree_body(dst_hbm, out_hbm, idx_v, ones_v, zbuf, acc_sh):
    c = lax.axis_index("c")
    s = lax.axis_index("s")
    wid = c * NS + s

    def fill_ones(i, carry):
        ones_v[i, :] = jnp.full((16,), 1.0, jnp.float32)
        return carry

    lax.fori_loop(0, CHUNK, fill_ones, 0)

    def fill_zero(i, carry):
        zbuf[i, :] = jnp.zeros((16,), jnp.float32)
        return carry

    lax.fori_loop(0, ZROWS, fill_zero, 0)

    for b in range(RPS // ZROWS):
        pltpu.sync_copy(zbuf, acc_sh.at[pl.ds(s * RPS + b * ZROWS, ZROWS)])
    plsc.subcore_barrier()

    pltpu.sync_copy(dst_hbm.at[wid], idx_v)

    def step(i, carry):
        pltpu.sync_copy(ones_v, acc_sh.at[idx_v.at[i]], add=True)
        return carry

    lax.fori_loop(0, NCH_DEG, step, 0)
    plsc.subcore_barrier()

    for b in range(RPS // ZROWS):
        base = s * RPS + b * ZROWS
        pltpu.sync_copy(acc_sh.at[pl.ds(base, ZROWS)], zbuf)
        pltpu.sync_copy(zbuf, out_hbm.at[c, pl.ds(base, ZROWS)])


def _sc_degree(dst_deg):
    return pl.kernel(
        _sc_degree_body,
        out_type=jax.ShapeDtypeStruct((NC, NP, 16), jnp.float32),
        mesh=_sc_mesh(),
        scratch_types=[
            pltpu.VMEM((NCH_DEG, CHUNK), jnp.int32),
            pltpu.VMEM((CHUNK, 16), jnp.float32),
            pltpu.VMEM((ZROWS, 16), jnp.float32),
            pltpu.VMEM_SHARED((NP, 16), jnp.float32),
        ],
    )(dst_deg)


def _sc_conv_body(g_hbm, src_hbm, dst_hbm, out_hbm, sidx_v, didx_v, rows_v,
                  zbuf, acc_sh):
    c = lax.axis_index("c")
    s = lax.axis_index("s")

    def fill_zero(i, carry):
        for l in range(HH // 16):
            zbuf[i, pl.ds(l * 16, 16)] = jnp.zeros((16,), jnp.float32)
        return carry

    lax.fori_loop(0, ZROWS, fill_zero, 0)

    for b in range(RPS // ZROWS):
        pltpu.sync_copy(zbuf, acc_sh.at[pl.ds(s * RPS + b * ZROWS, ZROWS)])
    plsc.subcore_barrier()

    pltpu.sync_copy(src_hbm.at[c, s], sidx_v)
    pltpu.sync_copy(dst_hbm.at[s], didx_v)

    def step(i, carry):
        pltpu.sync_copy(g_hbm.at[sidx_v.at[i]], rows_v)
        pltpu.sync_copy(rows_v, acc_sh.at[didx_v.at[i]], add=True)
        return carry

    lax.fori_loop(0, NCH_CONV, step, 0)
    plsc.subcore_barrier()

    for b in range(RPS // ZROWS):
        base = s * RPS + b * ZROWS
        pltpu.sync_copy(acc_sh.at[pl.ds(base, ZROWS)], zbuf)
        pltpu.sync_copy(zbuf, out_hbm.at[c, pl.ds(base, ZROWS)])


def _sc_conv(gflat, src4, dst3):
    return pl.kernel(
        _sc_conv_body,
        out_type=jax.ShapeDtypeStruct((NC, NP, HH), jnp.float32),
        mesh=_sc_mesh(),
        compiler_params=pltpu.CompilerParams(use_tc_tiling_on_sc=False),
        scratch_types=[
            pltpu.VMEM((NCH_CONV, CHUNK), jnp.int32),
            pltpu.VMEM((NCH_CONV, CHUNK), jnp.int32),
            pltpu.VMEM((CHUNK, HH), jnp.float32),
            pltpu.VMEM((ZROWS, HH), jnp.float32),
            pltpu.VMEM_SHARED((NP, HH), jnp.float32),
        ],
    )(gflat, src4, dst3)


def _tc_a_body(x_ref, w_ref, degp_ref, h_ref, g_ref, dinv_ref):
    h = jnp.dot(x_ref[...], w_ref[...], preferred_element_type=jnp.float32)
    dsum = degp_ref[0] + degp_ref[1]
    deg = jnp.sum(dsum, axis=1, keepdims=True) * (1.0 / 16.0) + 1.0
    dinv = 1.0 / jnp.sqrt(deg)
    g = h * dinv
    h_ref[...] = h
    g_ref[0, :, :] = g[:, :HH]
    g_ref[1, :, :] = g[:, HH:]
    dinv_ref[...] = dinv


def _tc_stage_a(x, W1, degp):
    return pl.pallas_call(
        _tc_a_body,
        grid=(N // BM,),
        in_specs=[
            pl.BlockSpec((BM, D), lambda i: (i, 0)),
            pl.BlockSpec((D, H), lambda i: (0, 0)),
            pl.BlockSpec((NC, BM, 16), lambda i: (0, i, 0)),
        ],
        out_specs=[
            pl.BlockSpec((BM, H), lambda i: (i, 0)),
            pl.BlockSpec((NC, BM, HH), lambda i: (0, i, 0)),
            pl.BlockSpec((BM, 1), lambda i: (i, 0)),
        ],
        out_shape=[
            jax.ShapeDtypeStruct((N, H), jnp.float32),
            jax.ShapeDtypeStruct((NC, N, HH), jnp.float32),
            jax.ShapeDtypeStruct((N, 1), jnp.float32),
        ],
    )(x, W1, degp)


def _b3(t):
    t2 = t * t
    t3 = t2 * t
    p0 = t3 * (1.0 / 6.0)
    p1 = (-3.0 * t3 + 12.0 * t2 - 12.0 * t + 4.0) * (1.0 / 6.0)
    p2 = (3.0 * t3 - 24.0 * t2 + 60.0 * t - 44.0) * (1.0 / 6.0)
    q = 4.0 - t
    p3 = q * q * q * (1.0 / 6.0)
    z = jnp.zeros_like(t)
    r = jnp.where((t >= 0.0) & (t < 1.0), p0, z)
    r = r + jnp.where((t >= 1.0) & (t < 2.0), p1, z)
    r = r + jnp.where((t >= 2.0) & (t < 3.0), p2, z)
    r = r + jnp.where((t >= 3.0) & (t < 4.0), p3, z)
    return r


def _tc_b_body(a_ref, h1_ref, dinv_ref, b1_ref, bw_ref, s_ref, w2_ref,
               h2_ref, g2_ref):
    dinv = dinv_ref[...]
    acc = jnp.concatenate([a_ref[0], a_ref[1]], axis=1)
    m = dinv * acc + (dinv * dinv) * h1_ref[...] + b1_ref[...]
    m = jnp.maximum(m, 0.0)
    sig = 1.0 / (1.0 + jnp.exp(-m))
    out = jnp.dot(m * sig, bw_ref[...], preferred_element_type=jnp.float32)
    u = m * 2.5 + 5.5
    for j in range(8):
        bj = _b3(u - jnp.float32(j))
        out = out + jnp.dot(bj, s_ref[j], preferred_element_type=jnp.float32)
    h2 = jnp.dot(out, w2_ref[...], preferred_element_type=jnp.float32)
    g2 = h2 * dinv
    h2_ref[...] = h2
    g2_ref[0, :, :] = g2[:, :HH]
    g2_ref[1, :, :] = g2[:, HH:]


def _tc_stage_b(acc1, h1, dinv, b1r, base_wT, S, W2):
    return pl.pallas_call(
        _tc_b_body,
        grid=(N // BM,),
        in_specs=[
            pl.BlockSpec((NC, BM, HH), lambda i: (0, i, 0)),
            pl.BlockSpec((BM, H), lambda i: (i, 0)),
            pl.BlockSpec((BM, 1), lambda i: (i, 0)),
            pl.BlockSpec((1, H), lambda i: (0, 0)),
            pl.BlockSpec((H, H), lambda i: (0, 0)),
            pl.BlockSpec((8, H, H), lambda i: (0, 0, 0)),
            pl.BlockSpec((H, H), lambda i: (0, 0)),
        ],
        out_specs=[
            pl.BlockSpec((BM, H), lambda i: (i, 0)),
            pl.BlockSpec((NC, BM, HH), lambda i: (0, i, 0)),
        ],
        out_shape=[
            jax.ShapeDtypeStruct((N, H), jnp.float32),
            jax.ShapeDtypeStruct((NC, N, HH), jnp.float32),
        ],
    )(acc1, h1, dinv, b1r, base_wT, S, W2)


def _tc_c_body(a_ref, h2_ref, dinv_ref, b2_ref, fw_ref, fb_ref, o_ref):
    dinv = dinv_ref[...]
    acc = jnp.concatenate([a_ref[0], a_ref[1]], axis=1)
    m = dinv * acc + (dinv * dinv) * h2_ref[...] + b2_ref[...]
    m = jnp.maximum(m, 0.0)
    logits = jnp.dot(m, fw_ref[...], preferred_element_type=jnp.float32)
    logits = logits + fb_ref[...]
    mx = jnp.max(logits, axis=1, keepdims=True)
    sh = logits - mx
    lse = jnp.log(jnp.sum(jnp.exp(sh), axis=1, keepdims=True))
    o_ref[...] = sh - lse


def _tc_stage_c(acc2, h2, dinv, b2r, fc_w, fc_br):
    return pl.pallas_call(
        _tc_c_body,
        grid=(N // BM,),
        in_specs=[
            pl.BlockSpec((NC, BM, HH), lambda i: (0, i, 0)),
            pl.BlockSpec((BM, H), lambda i: (i, 0)),
            pl.BlockSpec((BM, 1), lambda i: (i, 0)),
            pl.BlockSpec((1, H), lambda i: (0, 0)),
            pl.BlockSpec((H, C), lambda i: (0, 0)),
            pl.BlockSpec((1, C), lambda i: (0, 0)),
        ],
        out_specs=pl.BlockSpec((BM, C), lambda i: (i, 0)),
        out_shape=jax.ShapeDtypeStruct((N, C), jnp.float32),
    )(acc2, h2, dinv, b2r, fc_w, fc_br)


def kernel(x, edge_index, W1, b1, base_w, spline_w, W2, b2, fc_w, fc_b):
    src = edge_index[0]
    dst = edge_index[1]
    dst_deg = dst.reshape(NW, NCH_DEG, CHUNK)
    dst3 = dst.reshape(NS, NCH_CONV, CHUNK)
    src2 = src.reshape(NS, NCH_CONV, CHUNK)
    src4 = jnp.stack([src2, src2 + N])

    degp = _sc_degree(dst_deg)
    h1, g1, dinv = _tc_stage_a(x, W1, degp)
    acc1 = _sc_conv(g1.reshape(NC * N, HH), src4, dst3)

    base_wT = base_w.T
    S = jnp.transpose(spline_w, (2, 1, 0))
    h2, g2 = _tc_stage_b(acc1, h1, dinv, b1.reshape(1, H), base_wT, S, W2)

    acc2 = _sc_conv(g2.reshape(NC * N, HH), src4, dst3)
    return _tc_stage_c(acc2, h2, dinv, b2.reshape(1, H), fc_w,
                       fc_b.reshape(1, C))


# speedup vs baseline: 22.5986x; 1.7081x over previous
import jax
import jax.numpy as jnp
from jax import lax
from jax.experimental import pallas as pl
from jax.experimental.pallas import tpu as pltpu
from jax.experimental.pallas import tpu_sc as plsc

N = 10000
E = 320000
D = 128
H = 128
C = 64

NC = 2
NS = 16
NW = NC * NS
HH = H // NC

CHUNK_D = 80
NCH_DEG = (E // NW) // CHUNK_D
CHUNK = 125
NCH_CONV = (E // NS) // CHUNK

NP = 10240
RPS = NP // NS
ZROWS = 128

BM = 400


def _sc_mesh():
    return plsc.VectorSubcoreMesh(core_axis_name="c", subcore_axis_name="s")


def _sc_degree_body(dst_hbm, out_hbm, idx_v, ones_v, zbuf, acc_sh, sem):
    c = lax.axis_index("c")
    s = lax.axis_index("s")
    wid = c * NS + s

    def fill_ones(i, carry):
        ones_v[i, :] = jnp.full((16,), 1.0, jnp.float32)
        return carry

    lax.fori_loop(0, CHUNK_D, fill_ones, 0)

    def fill_zero(i, carry):
        zbuf[i, :] = jnp.zeros((16,), jnp.float32)
        return carry

    lax.fori_loop(0, ZROWS, fill_zero, 0)

    for b in range(RPS // ZROWS):
        pltpu.sync_copy(zbuf, acc_sh.at[pl.ds(s * RPS + b * ZROWS, ZROWS)])
    plsc.subcore_barrier()

    pltpu.sync_copy(dst_hbm.at[wid], idx_v)

    def fire(i, carry):
        pltpu.async_copy(ones_v, acc_sh.at[idx_v.at[i]], sem, add=True)
        return carry

    lax.fori_loop(0, NCH_DEG, fire, 0)

    def drain(i, carry):
        pltpu.make_async_copy(ones_v, acc_sh.at[idx_v.at[i]], sem).wait()
        return carry

    lax.fori_loop(0, NCH_DEG, drain, 0)
    plsc.subcore_barrier()

    for b in range(RPS // ZROWS):
        base = s * RPS + b * ZROWS
        pltpu.sync_copy(acc_sh.at[pl.ds(base, ZROWS)], zbuf)
        pltpu.sync_copy(zbuf, out_hbm.at[c, pl.ds(base, ZROWS)])


def _sc_degree(dst_deg):
    return pl.kernel(
        _sc_degree_body,
        out_type=jax.ShapeDtypeStruct((NC, NP, 16), jnp.float32),
        mesh=_sc_mesh(),
        scratch_types=[
            pltpu.VMEM((NCH_DEG, CHUNK_D), jnp.int32),
            pltpu.VMEM((CHUNK_D, 16), jnp.float32),
            pltpu.VMEM((ZROWS, 16), jnp.float32),
            pltpu.VMEM_SHARED((NP, 16), jnp.float32),
            pltpu.SemaphoreType.DMA,
        ],
    )(dst_deg)


def _sc_conv_body(g_hbm, src_hbm, dst_hbm, out_hbm, sidx_v, didx_v, rows0_v,
                  rows1_v, zbuf, acc_sh, sem0, sem1):
    c = lax.axis_index("c")
    s = lax.axis_index("s")

    def fill_zero(i, carry):
        for l in range(HH // 16):
            zbuf[i, pl.ds(l * 16, 16)] = jnp.zeros((16,), jnp.float32)
        return carry

    lax.fori_loop(0, ZROWS, fill_zero, 0)

    for b in range(RPS // ZROWS):
        pltpu.sync_copy(zbuf, acc_sh.at[pl.ds(s * RPS + b * ZROWS, ZROWS)])
    plsc.subcore_barrier()

    pltpu.sync_copy(src_hbm.at[c, s], sidx_v)
    pltpu.sync_copy(dst_hbm.at[s], didx_v)

    pltpu.async_copy(g_hbm.at[sidx_v.at[0]], rows0_v, sem0)

    def pair(p, carry):
        i = 2 * p
        pltpu.async_copy(g_hbm.at[sidx_v.at[i + 1]], rows1_v, sem1)
        pltpu.make_async_copy(g_hbm.at[sidx_v.at[i]], rows0_v, sem0).wait()
        pltpu.sync_copy(rows0_v, acc_sh.at[didx_v.at[i]], add=True)

        @pl.when(p < NCH_CONV // 2 - 1)
        def _():
            pltpu.async_copy(g_hbm.at[sidx_v.at[i + 2]], rows0_v, sem0)

        pltpu.make_async_copy(g_hbm.at[sidx_v.at[i + 1]], rows1_v, sem1).wait()
        pltpu.sync_copy(rows1_v, acc_sh.at[didx_v.at[i + 1]], add=True)
        return carry

    lax.fori_loop(0, NCH_CONV // 2, pair, 0)
    plsc.subcore_barrier()

    for b in range(RPS // ZROWS):
        base = s * RPS + b * ZROWS
        pltpu.sync_copy(acc_sh.at[pl.ds(base, ZROWS)], zbuf)
        pltpu.sync_copy(zbuf, out_hbm.at[c, pl.ds(base, ZROWS)])


def _sc_conv(gflat, src4, dst3):
    return pl.kernel(
        _sc_conv_body,
        out_type=jax.ShapeDtypeStruct((NC, NP, HH), jnp.float32),
        mesh=_sc_mesh(),
        compiler_params=pltpu.CompilerParams(use_tc_tiling_on_sc=False),
        scratch_types=[
            pltpu.VMEM((NCH_CONV, CHUNK), jnp.int32),
            pltpu.VMEM((NCH_CONV, CHUNK), jnp.int32),
            pltpu.VMEM((CHUNK, HH), jnp.float32),
            pltpu.VMEM((CHUNK, HH), jnp.float32),
            pltpu.VMEM((ZROWS, HH), jnp.float32),
            pltpu.VMEM_SHARED((NP, HH), jnp.float32),
            pltpu.SemaphoreType.DMA,
            pltpu.SemaphoreType.DMA,
        ],
    )(gflat, src4, dst3)


def _tc_a_body(x_ref, w_ref, degp_ref, h_ref, g_ref, dinv_ref):
    h = jnp.dot(x_ref[...], w_ref[...], preferred_element_type=jnp.float32)
    dsum = degp_ref[0] + degp_ref[1]
    deg = jnp.sum(dsum, axis=1, keepdims=True) * (1.0 / 16.0) + 1.0
    dinv = 1.0 / jnp.sqrt(deg)
    g = h * dinv
    h_ref[...] = h
    g_ref[0, :, :] = g[:, :HH]
    g_ref[1, :, :] = g[:, HH:]
    dinv_ref[...] = dinv


def _tc_stage_a(x, W1, degp):
    return pl.pallas_call(
        _tc_a_body,
        grid=(N // BM,),
        in_specs=[
            pl.BlockSpec((BM, D), lambda i: (i, 0)),
            pl.BlockSpec((D, H), lambda i: (0, 0)),
            pl.BlockSpec((NC, BM, 16), lambda i: (0, i, 0)),
        ],
        out_specs=[
            pl.BlockSpec((BM, H), lambda i: (i, 0)),
            pl.BlockSpec((NC, BM, HH), lambda i: (0, i, 0)),
            pl.BlockSpec((BM, 1), lambda i: (i, 0)),
        ],
        out_shape=[
            jax.ShapeDtypeStruct((N, H), jnp.float32),
            jax.ShapeDtypeStruct((NC, N, HH), jnp.float32),
            jax.ShapeDtypeStruct((N, 1), jnp.float32),
        ],
    )(x, W1, degp)


def _b3(t):
    t2 = t * t
    t3 = t2 * t
    p0 = t3 * (1.0 / 6.0)
    p1 = (-3.0 * t3 + 12.0 * t2 - 12.0 * t + 4.0) * (1.0 / 6.0)
    p2 = (3.0 * t3 - 24.0 * t2 + 60.0 * t - 44.0) * (1.0 / 6.0)
    q = 4.0 - t
    p3 = q * q * q * (1.0 / 6.0)
    z = jnp.zeros_like(t)
    r = jnp.where((t >= 0.0) & (t < 1.0), p0, z)
    r = r + jnp.where((t >= 1.0) & (t < 2.0), p1, z)
    r = r + jnp.where((t >= 2.0) & (t < 3.0), p2, z)
    r = r + jnp.where((t >= 3.0) & (t < 4.0), p3, z)
    return r


def _tc_b_body(a_ref, h1_ref, dinv_ref, b1_ref, bw_ref, s_ref, w2_ref,
               h2_ref, g2_ref):
    dinv = dinv_ref[...]
    acc = jnp.concatenate([a_ref[0], a_ref[1]], axis=1)
    m = dinv * acc + (dinv * dinv) * h1_ref[...] + b1_ref[...]
    m = jnp.maximum(m, 0.0)
    sig = 1.0 / (1.0 + jnp.exp(-m))
    out = jnp.dot(m * sig, bw_ref[...], preferred_element_type=jnp.float32)
    u = m * 2.5 + 5.5
    for j in range(8):
        bj = _b3(u - jnp.float32(j))
        out = out + jnp.dot(bj, s_ref[j], preferred_element_type=jnp.float32)
    h2 = jnp.dot(out, w2_ref[...], preferred_element_type=jnp.float32)
    g2 = h2 * dinv
    h2_ref[...] = h2
    g2_ref[0, :, :] = g2[:, :HH]
    g2_ref[1, :, :] = g2[:, HH:]


def _tc_stage_b(acc1, h1, dinv, b1r, base_wT, S, W2):
    return pl.pallas_call(
        _tc_b_body,
        grid=(N // BM,),
        in_specs=[
            pl.BlockSpec((NC, BM, HH), lambda i: (0, i, 0)),
            pl.BlockSpec((BM, H), lambda i: (i, 0)),
            pl.BlockSpec((BM, 1), lambda i: (i, 0)),
            pl.BlockSpec((1, H), lambda i: (0, 0)),
            pl.BlockSpec((H, H), lambda i: (0, 0)),
            pl.BlockSpec((8, H, H), lambda i: (0, 0, 0)),
            pl.BlockSpec((H, H), lambda i: (0, 0)),
        ],
        out_specs=[
            pl.BlockSpec((BM, H), lambda i: (i, 0)),
            pl.BlockSpec((NC, BM, HH), lambda i: (0, i, 0)),
        ],
        out_shape=[
            jax.ShapeDtypeStruct((N, H), jnp.float32),
            jax.ShapeDtypeStruct((NC, N, HH), jnp.float32),
        ],
    )(acc1, h1, dinv, b1r, base_wT, S, W2)


def _tc_c_body(a_ref, h2_ref, dinv_ref, b2_ref, fw_ref, fb_ref, o_ref):
    dinv = dinv_ref[...]
    acc = jnp.concatenate([a_ref[0], a_ref[1]], axis=1)
    m = dinv * acc + (dinv * dinv) * h2_ref[...] + b2_ref[...]
    m = jnp.maximum(m, 0.0)
    logits = jnp.dot(m, fw_ref[...], preferred_element_type=jnp.float32)
    logits = logits + fb_ref[...]
    mx = jnp.max(logits, axis=1, keepdims=True)
    sh = logits - mx
    lse = jnp.log(jnp.sum(jnp.exp(sh), axis=1, keepdims=True))
    o_ref[...] = sh - lse


def _tc_stage_c(acc2, h2, dinv, b2r, fc_w, fc_br):
    return pl.pallas_call(
        _tc_c_body,
        grid=(N // BM,),
        in_specs=[
            pl.BlockSpec((NC, BM, HH), lambda i: (0, i, 0)),
            pl.BlockSpec((BM, H), lambda i: (i, 0)),
            pl.BlockSpec((BM, 1), lambda i: (i, 0)),
            pl.BlockSpec((1, H), lambda i: (0, 0)),
            pl.BlockSpec((H, C), lambda i: (0, 0)),
            pl.BlockSpec((1, C), lambda i: (0, 0)),
        ],
        out_specs=pl.BlockSpec((BM, C), lambda i: (i, 0)),
        out_shape=jax.ShapeDtypeStruct((N, C), jnp.float32),
    )(acc2, h2, dinv, b2r, fc_w, fc_br)


def kernel(x, edge_index, W1, b1, base_w, spline_w, W2, b2, fc_w, fc_b):
    src = edge_index[0]
    dst = edge_index[1]
    dst_deg = dst.reshape(NW, NCH_DEG, CHUNK_D)
    dst3 = dst.reshape(NS, NCH_CONV, CHUNK)
    src2 = src.reshape(NS, NCH_CONV, CHUNK)
    src4 = jnp.stack([src2, src2 + N])

    degp = _sc_degree(dst_deg)
    h1, g1, dinv = _tc_stage_a(x, W1, degp)
    acc1 = _sc_conv(g1.reshape(NC * N, HH), src4, dst3)

    base_wT = base_w.T
    S = jnp.transpose(spline_w, (2, 1, 0))
    h2, g2 = _tc_stage_b(acc1, h1, dinv, b1.reshape(1, H), base_wT, S, W2)

    acc2 = _sc_conv(g2.reshape(NC * N, HH), src4, dst3)
    return _tc_stage_c(acc2, h2, dinv, b2.reshape(1, H), fc_w,
                       fc_b.reshape(1, C))
